# baseline (speedup 1.0000x reference)
import functools

import jax
import jax.numpy as jnp
from jax.experimental import pallas as pl

_B = 200


def _body(src_ref, nbr_ref, idx_ref, wagg_ref, bagg_ref, wself_ref, out_ref):
    seq = jnp.sum((idx_ref[...] != -1).astype(jnp.float32), axis=1)
    aggr = jnp.sum(nbr_ref[...], axis=1) / seq[:, None]
    nh = jnp.dot(aggr, wagg_ref[...], preferred_element_type=jnp.float32)
    nh = nh + bagg_ref[...]
    sh = jnp.dot(src_ref[...], wself_ref[...], preferred_element_type=jnp.float32)
    h = sh + nh
    out_ref[...] = jnp.where(h >= 0, h, 0.01 * h)


@jax.jit
def kernel(src_node_features, neighbor_node_features, neighbor_node_idx, W_agg, b_agg, W_self):
    n, k, d = neighbor_node_features.shape
    h = W_agg.shape[1]
    b = _B
    grid = (n // b,)
    bagg2d = b_agg.reshape(1, h)
    return pl.pallas_call(
        _body,
        grid=grid,
        in_specs=[
            pl.BlockSpec((b, d), lambda i: (i, 0)),
            pl.BlockSpec((b, k, d), lambda i: (i, 0, 0)),
            pl.BlockSpec((b, k), lambda i: (i, 0)),
            pl.BlockSpec((d, h), lambda i: (0, 0)),
            pl.BlockSpec((1, h), lambda i: (0, 0)),
            pl.BlockSpec((d, h), lambda i: (0, 0)),
        ],
        out_specs=pl.BlockSpec((b, h), lambda i: (i, 0)),
        out_shape=jax.ShapeDtypeStruct((n, h), jnp.float32),
    )(src_node_features, neighbor_node_features, neighbor_node_idx,
      W_agg, bagg2d, W_self)


# speedup vs baseline: 1.0133x; 1.0133x over previous
import functools

import jax
import jax.numpy as jnp
from jax.experimental import pallas as pl
from jax.experimental.pallas import tpu as pltpu

_B = 200


def _body(src_ref, nbr_ref, idx_ref, wagg_ref, bagg_ref, wself_ref, out_ref):
    seq = jnp.sum((idx_ref[...] != -1).astype(jnp.float32), axis=1)
    aggr = jnp.sum(nbr_ref[...], axis=1) / seq[:, None]
    nh = jnp.dot(aggr, wagg_ref[...], preferred_element_type=jnp.float32)
    nh = nh + bagg_ref[...]
    sh = jnp.dot(src_ref[...], wself_ref[...], preferred_element_type=jnp.float32)
    h = sh + nh
    out_ref[...] = jnp.where(h >= 0, h, 0.01 * h)


@jax.jit
def kernel(src_node_features, neighbor_node_features, neighbor_node_idx, W_agg, b_agg, W_self):
    n, k, d = neighbor_node_features.shape
    h = W_agg.shape[1]
    b = _B
    grid = (n // b,)
    bagg2d = b_agg.reshape(1, h)
    return pl.pallas_call(
        _body,
        grid=grid,
        in_specs=[
            pl.BlockSpec((b, d), lambda i: (i, 0)),
            pl.BlockSpec((b, k, d), lambda i: (i, 0, 0)),
            pl.BlockSpec((b, k), lambda i: (i, 0)),
            pl.BlockSpec((d, h), lambda i: (0, 0)),
            pl.BlockSpec((1, h), lambda i: (0, 0)),
            pl.BlockSpec((d, h), lambda i: (0, 0)),
        ],
        out_specs=pl.BlockSpec((b, h), lambda i: (i, 0)),
        out_shape=jax.ShapeDtypeStruct((n, h), jnp.float32),
        compiler_params=pltpu.CompilerParams(
            dimension_semantics=("parallel",),
        ),
    )(src_node_features, neighbor_node_features, neighbor_node_idx,
      W_agg, bagg2d, W_self)
